# baseline (speedup 1.0000x reference)
import jax
import jax.numpy as jnp
from jax import lax
from jax.experimental import pallas as pl
from jax.experimental.pallas import tpu as pltpu
from jax.experimental.pallas import tpu_sc as plsc

NC = 2
NS = 16
NW = NC * NS
L = 16
CH = 16


def _make_segsum(T, D, B):
  R = T // NW
  NCH = R // CH
  JS = D // L
  mesh = plsc.VectorSubcoreMesh(core_axis_name="c", subcore_axis_name="s")

  def body(flat_hbm, seg_hbm, partial_hbm, cnt_hbm,
           buf0, buf1, ids1d, acc, cntm, sem0, sem1):
    c = lax.axis_index("c")
    s = lax.axis_index("s")
    wid = c * NS + s
    base = wid * R

    pltpu.sync_copy(seg_hbm.at[pl.ds(base, R)], ids1d)

    z16 = jnp.zeros((L,), jnp.float32)
    ones16 = jnp.ones((L,), jnp.float32)

    def zero_acc(j, _):
      for r in range(B):
        acc[r, pl.ds(j * L, L)] = z16
      return 0
    lax.fori_loop(0, JS, zero_acc, 0)
    for r in range(B):
      cntm[r, :] = z16

    def load(i, buf, sem):
      pltpu.async_copy(flat_hbm.at[pl.ds(base + i * CH, CH)], buf, sem)

    def wait(buf, sem):
      pltpu.make_async_copy(flat_hbm.at[pl.ds(0, CH)], buf, sem).wait()

    chfull = jnp.full((L,), float(CH), jnp.float32)

    def process(chunk_i, buf):
      v = ids1d[pl.ds(chunk_i * CH, CH)]
      s_first = v[0]
      s_last = v[CH - 1]

      def uniform():
        plsc.addupdate(cntm.at[s_first], chfull)
        ST = 8

        @plsc.parallel_loop(0, JS // ST, 1, unroll=1)
        def _(m, _buf=buf, _s=s_first):
          bb = m * (ST * L)
          accs = [_buf[0, pl.ds(bb + t * L, L)] for t in range(ST)]
          for k in range(1, CH):
            for t in range(ST):
              accs[t] = accs[t] + _buf[k, pl.ds(bb + t * L, L)]
          for t in range(ST):
            plsc.addupdate(acc.at[_s, pl.ds(bb + t * L, L)], accs[t])

      def mixed():
        for k in range(0, CH, 2):
          s0 = v[k]
          s1 = v[k + 1]
          plsc.addupdate(cntm.at[s0], ones16)
          plsc.addupdate(cntm.at[s1], ones16)

          @plsc.parallel_loop(0, JS, 1, unroll=8)
          def _(j, _k=k, _s0=s0, _s1=s1, _buf=buf):
            sl = pl.ds(j * L, L)
            plsc.addupdate(acc.at[_s0, sl], _buf[_k, sl])
            sl2 = pl.ds((JS - 1 - j) * L, L)
            plsc.addupdate(acc.at[_s1, sl2], _buf[_k + 1, sl2])

      lax.cond(s_first == s_last, uniform, mixed)

    load(0, buf0, sem0)
    load(1, buf1, sem1)

    def loop(g, _):
      a = 2 * g
      wait(buf0, sem0)
      process(a, buf0)

      @pl.when(a + 2 < NCH)
      def _():
        load(a + 2, buf0, sem0)

      wait(buf1, sem1)
      process(a + 1, buf1)

      @pl.when(a + 3 < NCH)
      def _():
        load(a + 3, buf1, sem1)
      return 0
    lax.fori_loop(0, NCH // 2, loop, 0)

    pltpu.sync_copy(acc, partial_hbm.at[wid])
    pltpu.sync_copy(cntm, cnt_hbm.at[wid])

  return pl.kernel(
      body,
      out_type=(
          jax.ShapeDtypeStruct((NW, B, D), jnp.float32),
          jax.ShapeDtypeStruct((NW, B, L), jnp.float32),
      ),
      mesh=mesh,
      scratch_types=[
          pltpu.VMEM((CH, D), jnp.float32),
          pltpu.VMEM((CH, D), jnp.float32),
          pltpu.VMEM((R,), jnp.int32),
          pltpu.VMEM((B, D), jnp.float32),
          pltpu.VMEM((B, L), jnp.float32),
          pltpu.SemaphoreType.DMA,
          pltpu.SemaphoreType.DMA,
      ],
      name="segsum_sc",
  )


def _make_combine(D, B):
  def body(p_ref, c_ref, o_ref):
    sums = jnp.sum(p_ref[...], axis=0)
    cnts = jnp.sum(c_ref[...], axis=0)
    cvec = cnts[:, 0:1]
    o_ref[...] = sums / jnp.maximum(cvec, 1.0)

  return pl.pallas_call(
      body,
      out_shape=jax.ShapeDtypeStruct((B, D), jnp.float32),
      name="segmean_combine_tc",
  )


def kernel(flat, segment_ids):
  T, D = flat.shape
  B = 16
  seg32 = segment_ids.astype(jnp.int32)
  partial, cnt = _make_segsum(T, D, B)(flat, seg32)
  return _make_combine(D, B)(partial, cnt)


# speedup vs baseline: 1.8311x; 1.8311x over previous
import jax
import jax.numpy as jnp
from jax import lax
from jax.experimental import pallas as pl
from jax.experimental.pallas import tpu as pltpu
from jax.experimental.pallas import tpu_sc as plsc

NC = 2
NS = 16
NW = NC * NS
L = 16
CH = 32


def _make_segsum(T, D, B):
  R = T // NW
  NCH = R // CH
  JS = D // L
  mesh = plsc.VectorSubcoreMesh(core_axis_name="c", subcore_axis_name="s")

  def body(flat_hbm, seg_hbm, partial_hbm, cnt_hbm,
           buf0, buf1, ids1d, acc, cntm, sem0, sem1):
    c = lax.axis_index("c")
    s = lax.axis_index("s")
    wid = c * NS + s
    base = wid * R

    pltpu.sync_copy(seg_hbm.at[pl.ds(base, R)], ids1d)

    z16 = jnp.zeros((L,), jnp.float32)
    ones16 = jnp.ones((L,), jnp.float32)

    def zero_acc(j, _):
      for r in range(B):
        acc[r, pl.ds(j * L, L)] = z16
      return 0
    lax.fori_loop(0, JS, zero_acc, 0)
    for r in range(B):
      cntm[r, :] = z16

    def load(i, buf, sem):
      pltpu.async_copy(flat_hbm.at[pl.ds(base + i * CH, CH)], buf, sem)

    def wait(buf, sem):
      pltpu.make_async_copy(flat_hbm.at[pl.ds(0, CH)], buf, sem).wait()

    chfull = jnp.full((L,), float(CH), jnp.float32)

    def process(chunk_i, buf):
      v = ids1d[pl.ds(chunk_i * CH, CH)]
      s_first = v[0]
      s_last = v[CH - 1]

      def uniform():
        plsc.addupdate(cntm.at[s_first], chfull)
        ST = 4

        @plsc.parallel_loop(0, JS // ST, 1, unroll=2)
        def _(m, _buf=buf, _s=s_first):
          bb = m * (ST * L)
          accs = [_buf[0, pl.ds(bb + t * L, L)] for t in range(ST)]
          for k in range(1, CH):
            for t in range(ST):
              accs[t] = accs[t] + _buf[k, pl.ds(bb + t * L, L)]
          for t in range(ST):
            plsc.addupdate(acc.at[_s, pl.ds(bb + t * L, L)], accs[t])

      def mixed():
        for k in range(0, CH, 2):
          s0 = v[k]
          s1 = v[k + 1]
          plsc.addupdate(cntm.at[s0], ones16)
          plsc.addupdate(cntm.at[s1], ones16)

          @plsc.parallel_loop(0, JS, 1, unroll=8)
          def _(j, _k=k, _s0=s0, _s1=s1, _buf=buf):
            sl = pl.ds(j * L, L)
            plsc.addupdate(acc.at[_s0, sl], _buf[_k, sl])
            sl2 = pl.ds((JS - 1 - j) * L, L)
            plsc.addupdate(acc.at[_s1, sl2], _buf[_k + 1, sl2])

      lax.cond(s_first == s_last, uniform, mixed)

    load(0, buf0, sem0)
    load(1, buf1, sem1)

    def loop(g, _):
      a = 2 * g
      wait(buf0, sem0)
      process(a, buf0)

      @pl.when(a + 2 < NCH)
      def _():
        load(a + 2, buf0, sem0)

      wait(buf1, sem1)
      process(a + 1, buf1)

      @pl.when(a + 3 < NCH)
      def _():
        load(a + 3, buf1, sem1)
      return 0
    lax.fori_loop(0, NCH // 2, loop, 0)

    pltpu.sync_copy(acc, partial_hbm.at[wid])
    pltpu.sync_copy(cntm, cnt_hbm.at[wid])

  return pl.kernel(
      body,
      out_type=(
          jax.ShapeDtypeStruct((NW, B, D), jnp.float32),
          jax.ShapeDtypeStruct((NW, B, L), jnp.float32),
      ),
      mesh=mesh,
      scratch_types=[
          pltpu.VMEM((CH, D), jnp.float32),
          pltpu.VMEM((CH, D), jnp.float32),
          pltpu.VMEM((R,), jnp.int32),
          pltpu.VMEM((B, D), jnp.float32),
          pltpu.VMEM((B, L), jnp.float32),
          pltpu.SemaphoreType.DMA,
          pltpu.SemaphoreType.DMA,
      ],
      name="segsum_sc",
  )


def _make_combine(D, B):
  def body(p_ref, c_ref, o_ref):
    sums = jnp.sum(p_ref[...], axis=0)
    cnts = jnp.sum(c_ref[...], axis=0)
    cvec = cnts[:, 0:1]
    o_ref[...] = sums / jnp.maximum(cvec, 1.0)

  return pl.pallas_call(
      body,
      out_shape=jax.ShapeDtypeStruct((B, D), jnp.float32),
      name="segmean_combine_tc",
  )


def kernel(flat, segment_ids):
  T, D = flat.shape
  B = 16
  seg32 = segment_ids.astype(jnp.int32)
  partial, cnt = _make_segsum(T, D, B)(flat, seg32)
  return _make_combine(D, B)(partial, cnt)
